# baseline (speedup 1.0000x reference)
import functools

import jax
import jax.numpy as jnp
from jax import lax
from jax.experimental import pallas as pl
from jax.experimental.pallas import tpu as pltpu
from jax.experimental.pallas import tpu_sc as plsc

N = 10000
DIM = 128
NC = 2
NS = 16
NW = NC * NS
L = 16
CHUNK = 128
NSPLIT = 4
SUB = CHUNK // NSPLIT
RPT = 624
TAIL = N - NS * RPT


def _sweep_body(z, colr, rowr, ewr, zrows, out,
                ca0, ca1, ca2, ca3, rva, ewa, rwa,
                cb0, cb1, cb2, cb3, rvb, ewb, rwb,
                acc, ga0, ga1, ga2, ga3, gb0, gb1, gb2, gb3,
                isa, isb, *, nchunk):
    colv = ((ca0, ca1, ca2, ca3), (cb0, cb1, cb2, cb3))
    rowv = (rva, rvb)
    ewv = (ewa, ewb)
    rows = (rwa, rwb)
    gs = ((ga0, ga1, ga2, ga3), (gb0, gb1, gb2, gb3))
    isem = (isa, isb)
    c = lax.axis_index("c")
    s = lax.axis_index("s")
    wid = c * NS + s
    tbase = wid * nchunk * CHUNK

    def idx_start(i, p):
        base = pl.multiple_of(tbase + i * CHUNK, 8)
        for k in range(NSPLIT):
            pltpu.async_copy(colr.at[pl.ds(base + k * SUB, SUB)],
                             colv[p][k], isem[p])
        pltpu.async_copy(rowr.at[pl.ds(base, CHUNK)], rowv[p], isem[p])
        pltpu.async_copy(ewr.at[pl.ds(base, CHUNK)], ewv[p], isem[p])

    def idx_wait(p):
        for k in range(NSPLIT):
            pltpu.make_async_copy(colr.at[pl.ds(0, SUB)], colv[p][k],
                                  isem[p]).wait()
        pltpu.make_async_copy(rowr.at[pl.ds(0, CHUNK)], rowv[p],
                              isem[p]).wait()
        pltpu.make_async_copy(ewr.at[pl.ds(0, CHUNK)], ewv[p],
                              isem[p]).wait()

    def gather_start(p):
        for k in range(NSPLIT):
            pltpu.async_copy(z.at[colv[p][k]],
                             rows[p].at[pl.ds(k * SUB, SUB)], gs[p][k])

    def gather_wait(p):
        for k in range(NSPLIT):
            pltpu.make_async_copy(z.at[colv[p][k]],
                                  rows[p].at[pl.ds(k * SUB, SUB)],
                                  gs[p][k]).wait()

    idx_start(0, 0)
    idx_wait(0)
    gather_start(0)
    idx_start(1, 1)

    zbase = pl.multiple_of(s * RPT, 8)
    pltpu.sync_copy(zrows.at[pl.ds(0, RPT)], acc.at[pl.ds(zbase, RPT)])

    @pl.when(s == NS - 1)
    def _():
        pltpu.sync_copy(zrows.at[pl.ds(0, TAIL)], acc.at[pl.ds(NS * RPT, TAIL)])

    plsc.subcore_barrier()

    def pair(gp, carry):
        for p in range(2):
            i = gp * 2 + p
            q = 1 - p
            gather_wait(p)

            @pl.when(i + 1 < nchunk)
            def _(p=p, q=q):
                idx_wait(q)
                gather_start(q)

            def grp(g2_, carry2, _p=p):
                wvec = ewv[_p][pl.ds(g2_ * L, L)]
                for t in range(L):
                    wv = jnp.full((L,), wvec[t], jnp.float32)
                    e = g2_ * L + t
                    for j in range(DIM // L):
                        rows[_p][e, pl.ds(j * L, L)] = (
                            rows[_p][e, pl.ds(j * L, L)] * wv)
                return carry2

            lax.fori_loop(0, CHUNK // L, grp, 0)

            pltpu.sync_copy(rows[p], acc.at[rowv[p]], add=True)

            @pl.when(i + 2 < nchunk)
            def _(i=i, p=p):
                idx_start(i + 2, p)

        return carry

    lax.fori_loop(0, nchunk // 2, pair, 0)

    plsc.subcore_barrier()
    dbase = pl.multiple_of(s * RPT, 8)
    pltpu.sync_copy(acc.at[pl.ds(dbase, RPT)], out.at[c, pl.ds(dbase, RPT)])

    @pl.when(s == NS - 1)
    def _():
        pltpu.sync_copy(acc.at[pl.ds(NS * RPT, TAIL)],
                        out.at[c, pl.ds(NS * RPT, TAIL)])


def _make_sweep(nchunk):
    mesh = plsc.VectorSubcoreMesh(core_axis_name="c", subcore_axis_name="s",
                                  num_cores=NC, num_subcores=NS)
    return pl.kernel(
        functools.partial(_sweep_body, nchunk=nchunk),
        out_type=jax.ShapeDtypeStruct((NC, N, DIM), jnp.float32),
        mesh=mesh,
        scratch_types=([pltpu.VMEM((SUB,), jnp.int32)] * NSPLIT + [
            pltpu.VMEM((CHUNK,), jnp.int32),
            pltpu.VMEM((CHUNK,), jnp.float32),
            pltpu.VMEM((CHUNK, DIM), jnp.float32),
        ]) * 2 + [
            pltpu.VMEM_SHARED((N, DIM), jnp.float32),
        ] + [pltpu.SemaphoreType.DMA] * (2 * NSPLIT + 2),
    )


def _pre_body(x_ref, z_ref):
    x = x_ref[...]
    head = x[:, 0:1]
    tail = x[:, 1:]
    xk = tail / head
    n2 = jnp.clip(jnp.sum(xk * xk, axis=1, keepdims=True), 0.0, 0.9)
    lamb = 1.0 / jnp.sqrt(1.0 - n2)
    z_ref[...] = jnp.concatenate([lamb, lamb * xk], axis=1)


def _combine(p):
    a = p[0] + p[1]
    a0 = a[:, 0:1]
    inv = jnp.where(a0 != 0.0, 1.0 / a0, 0.0)
    km = a[:, 1:] * inv
    n2 = jnp.clip(jnp.sum(km * km, axis=1, keepdims=True), 0.0, 0.9)
    lamb = 1.0 / jnp.sqrt(1.0 - n2)
    pm = km * (lamb / (lamb + 1.0))
    alpha = 1.6732632423543772
    scale = 1.0507009873554805
    sp = scale * jnp.where(pm > 0, pm, alpha * (jnp.exp(pm) - 1.0))
    n2s = jnp.sum(sp * sp, axis=1, keepdims=True)
    denom = jnp.maximum(1.0 - n2s, 1e-6)
    xr = 2.0 * sp / denom
    headn = jnp.sqrt(1.0 + jnp.sum(xr * xr, axis=1, keepdims=True))
    return xr, headn


def _mid_body(p_ref, z_ref):
    xr, headn = _combine(p_ref[...])
    xk = xr / headn
    n2 = jnp.clip(jnp.sum(xk * xk, axis=1, keepdims=True), 0.0, 0.9)
    lamb = 1.0 / jnp.sqrt(1.0 - n2)
    z_ref[...] = jnp.concatenate([lamb, lamb * xk], axis=1)


def _post_body(p_ref, o_ref):
    xr, headn = _combine(p_ref[...])
    o_ref[...] = jnp.concatenate([headn, xr], axis=1)


_BLK = 1000


def _dense_pre(x):
    return pl.pallas_call(
        _pre_body,
        grid=(N // _BLK,),
        in_specs=[pl.BlockSpec((_BLK, DIM), lambda i: (i, 0))],
        out_specs=pl.BlockSpec((_BLK, DIM), lambda i: (i, 0)),
        out_shape=jax.ShapeDtypeStruct((N, DIM), jnp.float32),
    )(x)


def _dense_stage(body, p):
    return pl.pallas_call(
        body,
        grid=(N // _BLK,),
        in_specs=[pl.BlockSpec((NC, _BLK, DIM), lambda i: (0, i, 0))],
        out_specs=pl.BlockSpec((_BLK, DIM), lambda i: (i, 0)),
        out_shape=jax.ShapeDtypeStruct((N, DIM), jnp.float32),
    )(p)


def kernel(x, edge_index, edge_weight, msg_weight):
    del msg_weight
    row = edge_index[0]
    col = edge_index[1]
    e = edge_weight.shape[0]
    nchunk = -(-e // (NW * CHUNK))
    nchunk = -(-nchunk // 2) * 2
    pad = NW * nchunk * CHUNK - e
    if pad:
        row = jnp.pad(row, (0, pad))
        col = jnp.pad(col, (0, pad))
        edge_weight = jnp.pad(edge_weight, (0, pad))
    zrows = jnp.zeros((RPT, DIM), jnp.float32)

    sweep = _make_sweep(nchunk)
    z = _dense_pre(x)
    p = sweep(z, col, row, edge_weight, zrows)
    z = _dense_stage(_mid_body, p)
    p = sweep(z, col, row, edge_weight, zrows)
    return _dense_stage(_post_body, p)
